# Initial kernel scaffold; baseline (speedup 1.0000x reference)
import jax
import jax.numpy as jnp
from jax.experimental import pallas as pl


def kernel(priors, potential, src_nodes, dst_nodes, rev_edges):
    raise NotImplementedError("write your pallas kernel here")


# speedup vs baseline: 2.0199x; 2.0199x over previous
import functools

import jax
import jax.numpy as jnp
from jax import lax
from jax.experimental import pallas as pl
from jax.experimental.pallas import tpu as pltpu
from jax.experimental.pallas import tpu_sc as plsc

NUM_ITERS = 4
THRESH = 1e-08

NSC = 2
NSUB = 16
NW = NSC * NSUB
CHUNK = 128


def _mesh():
    return plsc.VectorSubcoreMesh(
        core_axis_name="c", subcore_axis_name="s",
        num_cores=NSC, num_subcores=NSUB)


def _worker_id():
    return lax.axis_index("s") * NSC + lax.axis_index("c")


def _sc_gather(table, idx2d, num_edges, num_classes):
    nrows = num_edges // CHUNK
    per_w = nrows // NW
    extra = nrows - per_w * NW

    @functools.partial(
        pl.kernel,
        mesh=_mesh(),
        out_type=jax.ShapeDtypeStruct((num_edges, num_classes), jnp.float32),
        scratch_types=[
            pltpu.VMEM((CHUNK,), jnp.int32),
            pltpu.VMEM((CHUNK, num_classes), jnp.float32),
            pltpu.SemaphoreType.DMA,
        ],
        compiler_params=pltpu.CompilerParams(use_tc_tiling_on_sc=False),
    )
    def k(table_hbm, idx_hbm, out_hbm, idx_v, rows_v, sem):
        wid = _worker_id()
        nloc = per_w + jnp.where(wid < extra, 1, 0)

        def step(j, carry):
            r = wid + j * NW
            pltpu.sync_copy(idx_hbm.at[r], idx_v)
            pltpu.async_copy(table_hbm.at[idx_v], rows_v, sem).wait()
            pltpu.sync_copy(rows_v, out_hbm.at[pl.ds(r * CHUNK, CHUNK)])
            return carry

        lax.fori_loop(0, nloc, step, 0)

    return k(table, idx2d)


def _sc_scatter_add(vals, idx2d, num_nodes, num_classes):
    num_edges = vals.shape[0]
    nrows = num_edges // CHUNK
    per_w = nrows // NW
    extra = nrows - per_w * NW
    zrows = 125
    per_sub = num_nodes // NSUB

    @functools.partial(
        pl.kernel,
        mesh=_mesh(),
        out_type=jax.ShapeDtypeStruct((NSC, num_nodes, num_classes),
                                      jnp.float32),
        scratch_types=[
            pltpu.VMEM((CHUNK,), jnp.int32),
            pltpu.VMEM((CHUNK, num_classes), jnp.float32),
            pltpu.VMEM((zrows, num_classes), jnp.float32),
            pltpu.VMEM_SHARED((num_nodes, num_classes), jnp.float32),
        ],
        compiler_params=pltpu.CompilerParams(use_tc_tiling_on_sc=False),
    )
    def k(vals_hbm, idx_hbm, out_hbm, idx_v, rows_v, zbuf_v, acc_sh):
        cid = lax.axis_index("c")
        sid = lax.axis_index("s")
        wid = sid * NSC + cid

        def zfill(i, carry):
            zbuf_v[i, :] = jnp.zeros((num_classes,), jnp.float32)
            return carry
        lax.fori_loop(0, zrows, zfill, 0)

        def zcopy(kk, carry):
            pltpu.sync_copy(
                zbuf_v, acc_sh.at[pl.ds(sid * per_sub + kk * zrows, zrows)])
            return carry
        lax.fori_loop(0, per_sub // zrows, zcopy, 0)
        plsc.subcore_barrier()

        nloc = per_w + jnp.where(wid < extra, 1, 0)

        def step(j, carry):
            r = wid + j * NW
            pltpu.sync_copy(idx_hbm.at[r], idx_v)
            pltpu.sync_copy(vals_hbm.at[pl.ds(r * CHUNK, CHUNK)], rows_v)
            pltpu.sync_copy(rows_v, acc_sh.at[idx_v], add=True)
            return carry
        lax.fori_loop(0, nloc, step, 0)
        plsc.subcore_barrier()

        pltpu.sync_copy(
            acc_sh.at[pl.ds(sid * per_sub, per_sub)],
            out_hbm.at[cid, pl.ds(sid * per_sub, per_sub)])

    return k(vals, idx2d)


def _tc_edge_math(bsrc, lnrev, potential, first):
    num_edges, num_classes = bsrc.shape
    be = 4000
    grid = num_edges // be

    def body(*refs):
        if first:
            bsrc_ref, pot_ref, out_ref = refs
            r = bsrc_ref[...]
        else:
            bsrc_ref, lnrev_ref, pot_ref, out_ref = refs
            r = bsrc_ref[...] * jnp.exp(-lnrev_ref[...])
        t = jnp.dot(r, pot_ref[...], preferred_element_type=jnp.float32)
        n = t / jnp.sum(t, axis=1, keepdims=True)
        out_ref[...] = jnp.log(n)

    espec = pl.BlockSpec((be, num_classes), lambda i: (i, 0))
    pspec = pl.BlockSpec((num_classes, num_classes), lambda i: (0, 0))
    in_specs = [espec, pspec] if first else [espec, espec, pspec]
    args = (bsrc, potential) if first else (bsrc, lnrev, potential)
    return pl.pallas_call(
        body,
        grid=(grid,),
        in_specs=in_specs,
        out_specs=espec,
        out_shape=jax.ShapeDtypeStruct((num_edges, num_classes), jnp.float32),
    )(*args)


def _tc_softmax(parts, priors, old_beliefs):
    num_nodes, num_classes = priors.shape
    bn = 5000
    grid = num_nodes // bn

    def body(parts_ref, pri_ref, old_ref, bel_ref, dmax_ref):
        logits = jnp.log(pri_ref[...]) + parts_ref[0] + parts_ref[1]
        m = jnp.max(logits, axis=1, keepdims=True)
        e = jnp.exp(logits - m)
        b = e / jnp.sum(e, axis=1, keepdims=True)
        bel_ref[...] = b
        d = jnp.max(jnp.abs(b - old_ref[...]))
        dmax_ref[...] = jnp.full((8, 128), d, jnp.float32)

    nspec = pl.BlockSpec((bn, num_classes), lambda i: (i, 0))
    return pl.pallas_call(
        body,
        grid=(grid,),
        in_specs=[
            pl.BlockSpec((NSC, bn, num_classes), lambda i: (0, i, 0)),
            nspec,
            nspec,
        ],
        out_specs=[nspec, pl.BlockSpec((8, 128), lambda i: (i, 0))],
        out_shape=[
            jax.ShapeDtypeStruct((num_nodes, num_classes), jnp.float32),
            jax.ShapeDtypeStruct((grid * 8, 128), jnp.float32),
        ],
    )(parts, priors, old_beliefs)


def kernel(priors, potential, src_nodes, dst_nodes, rev_edges):
    num_edges = src_nodes.shape[0]
    num_nodes, num_classes = priors.shape
    nrows = num_edges // CHUNK

    src2 = src_nodes.reshape(nrows, CHUNK)
    dst2 = dst_nodes.reshape(nrows, CHUNK)
    rev2 = rev_edges.reshape(nrows, CHUNK)

    beliefs = priors
    ln = None
    done = jnp.array(False)
    for it in range(NUM_ITERS):
        bsrc = _sc_gather(beliefs, src2, num_edges, num_classes)
        if it == 0:
            ln_new = _tc_edge_math(bsrc, None, potential, first=True)
        else:
            lnrev = _sc_gather(ln, rev2, num_edges, num_classes)
            ln_new = _tc_edge_math(bsrc, lnrev, potential, first=False)
        parts = _sc_scatter_add(ln_new, dst2, num_nodes, num_classes)
        b_new, bmax = _tc_softmax(parts, priors, beliefs)
        diff = jnp.max(bmax)
        ln = ln_new if it == 0 else jnp.where(done, ln, ln_new)
        beliefs = jnp.where(done, beliefs, b_new)
        done = jnp.logical_or(done, diff < THRESH)
    return beliefs


# speedup vs baseline: 2.5457x; 1.2603x over previous
import functools

import jax
import jax.numpy as jnp
from jax import lax
from jax.experimental import pallas as pl
from jax.experimental.pallas import tpu as pltpu
from jax.experimental.pallas import tpu_sc as plsc

NUM_ITERS = 4
THRESH = 1e-08

NSC = 2
NSUB = 16
NW = NSC * NSUB
CHUNK = 125
K = 20
GROUP = K * CHUNK
KS = 10
GROUPS = KS * CHUNK


def _mesh():
    return plsc.VectorSubcoreMesh(
        core_axis_name="c", subcore_axis_name="s",
        num_cores=NSC, num_subcores=NSUB)


def _sc_params():
    return pltpu.CompilerParams(use_tc_tiling_on_sc=False)


def _sc_gather(table, idx2d, num_edges, num_classes):
    nrows = num_edges // CHUNK
    per_w = nrows // NW
    ng = per_w // K
    npairs = ng // 2

    @functools.partial(
        pl.kernel,
        mesh=_mesh(),
        out_type=jax.ShapeDtypeStruct((num_edges, num_classes), jnp.float32),
        scratch_types=[
            pltpu.VMEM((per_w, CHUNK), jnp.int32),
            pltpu.VMEM((2, GROUP, num_classes), jnp.float32),
            pltpu.SemaphoreType.DMA,
            pltpu.SemaphoreType.DMA,
            pltpu.SemaphoreType.DMA,
        ],
        compiler_params=_sc_params(),
    )
    def k(table_hbm, idx_hbm, out_hbm, idx_v, rows_v, gsem, osem0, osem1):
        wid = lax.axis_index("s") * NSC + lax.axis_index("c")
        row0 = wid * per_w
        ebase = row0 * CHUNK

        pltpu.sync_copy(idx_hbm.at[pl.ds(row0, per_w)], idx_v)

        def out_slice(g):
            return out_hbm.at[pl.ds(ebase + g * GROUP, GROUP)]

        def do_group(g, b, osem, first):
            if not first:
                pltpu.make_async_copy(rows_v.at[b], out_slice(g - 2),
                                      osem).wait()

            def fire(kk, c):
                pltpu.async_copy(
                    table_hbm.at[idx_v.at[g * K + kk]],
                    rows_v.at[b, pl.ds(kk * CHUNK, CHUNK)], gsem)
                return c
            lax.fori_loop(0, K, fire, 0)

            def drain(kk, c):
                pltpu.make_async_copy(
                    table_hbm.at[pl.ds(0, CHUNK)],
                    rows_v.at[b, pl.ds(kk * CHUNK, CHUNK)], gsem).wait()
                return c
            lax.fori_loop(0, K, drain, 0)

            pltpu.async_copy(rows_v.at[b], out_slice(g), osem)

        do_group(0, 0, osem0, first=True)
        do_group(1, 1, osem1, first=True)

        def pair(p, c):
            do_group(2 * p, 0, osem0, first=False)
            do_group(2 * p + 1, 1, osem1, first=False)
            return c
        lax.fori_loop(1, npairs, pair, 0)

        pltpu.make_async_copy(rows_v.at[0], out_slice(ng - 2), osem0).wait()
        pltpu.make_async_copy(rows_v.at[1], out_slice(ng - 1), osem1).wait()

    return k(table, idx2d)


def _sc_scatter_add(vals, idx2d, num_nodes, num_classes):
    num_edges = vals.shape[0]
    nrows = num_edges // CHUNK
    per_w = nrows // NW
    ng = per_w // KS
    npairs = ng // 2
    zrows = 125
    per_sub = num_nodes // NSUB

    @functools.partial(
        pl.kernel,
        mesh=_mesh(),
        out_type=jax.ShapeDtypeStruct((NSC, num_nodes, num_classes),
                                      jnp.float32),
        scratch_types=[
            pltpu.VMEM((per_w, CHUNK), jnp.int32),
            pltpu.VMEM((2, GROUPS, num_classes), jnp.float32),
            pltpu.VMEM((zrows, num_classes), jnp.float32),
            pltpu.VMEM_SHARED((num_nodes, num_classes), jnp.float32),
            pltpu.SemaphoreType.DMA,
            pltpu.SemaphoreType.DMA,
            pltpu.SemaphoreType.DMA,
        ],
        compiler_params=_sc_params(),
    )
    def k(vals_hbm, idx_hbm, out_hbm, idx_v, rows_v, zbuf_v, acc_sh,
          vsem0, vsem1, ssem):
        cid = lax.axis_index("c")
        sid = lax.axis_index("s")
        wid = sid * NSC + cid
        row0 = wid * per_w
        ebase = row0 * CHUNK

        def zfill(i, c):
            zbuf_v[i, :] = jnp.zeros((num_classes,), jnp.float32)
            return c
        lax.fori_loop(0, zrows, zfill, 0)

        def zcopy(kk, c):
            pltpu.sync_copy(
                zbuf_v, acc_sh.at[pl.ds(sid * per_sub + kk * zrows, zrows)])
            return c
        lax.fori_loop(0, per_sub // zrows, zcopy, 0)

        pltpu.sync_copy(idx_hbm.at[pl.ds(row0, per_w)], idx_v)
        plsc.subcore_barrier()

        def vals_slice(g):
            return vals_hbm.at[pl.ds(ebase + g * GROUPS, GROUPS)]

        def do_group(g, b, vsem, vsem_next, last):
            pltpu.make_async_copy(vals_slice(g), rows_v.at[b], vsem).wait()
            if not last:
                pltpu.async_copy(vals_slice(g + 1), rows_v.at[1 - b],
                                 vsem_next)

            def fire(kk, c):
                pltpu.async_copy(
                    rows_v.at[b, pl.ds(kk * CHUNK, CHUNK)],
                    acc_sh.at[idx_v.at[g * KS + kk]], ssem, add=True)
                return c
            lax.fori_loop(0, KS, fire, 0)

            def drain(kk, c):
                pltpu.make_async_copy(
                    vals_hbm.at[pl.ds(0, CHUNK)],
                    rows_v.at[b, pl.ds(kk * CHUNK, CHUNK)], ssem).wait()
                return c
            lax.fori_loop(0, KS, drain, 0)

        pltpu.async_copy(vals_slice(0), rows_v.at[0], vsem0)

        def pair(p, c):
            g = 2 * p
            do_group(g, 0, vsem0, vsem1, last=False)
            do_group(g + 1, 1, vsem1, vsem0, last=False)
            return c
        lax.fori_loop(0, npairs - 1, pair, 0)
        do_group(ng - 2, 0, vsem0, vsem1, last=False)
        do_group(ng - 1, 1, vsem1, vsem0, last=True)

        plsc.subcore_barrier()
        pltpu.sync_copy(
            acc_sh.at[pl.ds(sid * per_sub, per_sub)],
            out_hbm.at[cid, pl.ds(sid * per_sub, per_sub)])

    return k(vals, idx2d)


def _tc_edge_math(bsrc, lnrev, potential, first):
    num_edges, num_classes = bsrc.shape
    be = 4000
    grid = num_edges // be

    def body(*refs):
        if first:
            bsrc_ref, pot_ref, out_ref = refs
            r = bsrc_ref[...]
        else:
            bsrc_ref, lnrev_ref, pot_ref, out_ref = refs
            r = bsrc_ref[...] * jnp.exp(-lnrev_ref[...])
        t = jnp.dot(r, pot_ref[...], preferred_element_type=jnp.float32)
        n = t / jnp.sum(t, axis=1, keepdims=True)
        out_ref[...] = jnp.log(n)

    espec = pl.BlockSpec((be, num_classes), lambda i: (i, 0))
    pspec = pl.BlockSpec((num_classes, num_classes), lambda i: (0, 0))
    in_specs = [espec, pspec] if first else [espec, espec, pspec]
    args = (bsrc, potential) if first else (bsrc, lnrev, potential)
    return pl.pallas_call(
        body,
        grid=(grid,),
        in_specs=in_specs,
        out_specs=espec,
        out_shape=jax.ShapeDtypeStruct((num_edges, num_classes), jnp.float32),
    )(*args)


def _tc_softmax(parts, priors, old_beliefs):
    num_nodes, num_classes = priors.shape
    bn = 5000
    grid = num_nodes // bn

    def body(parts_ref, pri_ref, old_ref, bel_ref, dmax_ref):
        logits = jnp.log(pri_ref[...]) + parts_ref[0] + parts_ref[1]
        m = jnp.max(logits, axis=1, keepdims=True)
        e = jnp.exp(logits - m)
        b = e / jnp.sum(e, axis=1, keepdims=True)
        bel_ref[...] = b
        d = jnp.max(jnp.abs(b - old_ref[...]))
        dmax_ref[...] = jnp.full((8, 128), d, jnp.float32)

    nspec = pl.BlockSpec((bn, num_classes), lambda i: (i, 0))
    return pl.pallas_call(
        body,
        grid=(grid,),
        in_specs=[
            pl.BlockSpec((NSC, bn, num_classes), lambda i: (0, i, 0)),
            nspec,
            nspec,
        ],
        out_specs=[nspec, pl.BlockSpec((8, 128), lambda i: (i, 0))],
        out_shape=[
            jax.ShapeDtypeStruct((num_nodes, num_classes), jnp.float32),
            jax.ShapeDtypeStruct((grid * 8, 128), jnp.float32),
        ],
    )(parts, priors, old_beliefs)


def kernel(priors, potential, src_nodes, dst_nodes, rev_edges):
    num_edges = src_nodes.shape[0]
    num_nodes, num_classes = priors.shape
    nrows = num_edges // CHUNK

    src2 = src_nodes.reshape(nrows, CHUNK)
    dst2 = dst_nodes.reshape(nrows, CHUNK)
    rev2 = rev_edges.reshape(nrows, CHUNK)

    beliefs = priors
    ln = None
    done = jnp.array(False)
    for it in range(NUM_ITERS):
        bsrc = _sc_gather(beliefs, src2, num_edges, num_classes)
        if it == 0:
            ln_new = _tc_edge_math(bsrc, None, potential, first=True)
        else:
            lnrev = _sc_gather(ln, rev2, num_edges, num_classes)
            ln_new = _tc_edge_math(bsrc, lnrev, potential, first=False)
        parts = _sc_scatter_add(ln_new, dst2, num_nodes, num_classes)
        b_new, bmax = _tc_softmax(parts, priors, beliefs)
        diff = jnp.max(bmax)
        ln = ln_new if it == 0 else jnp.where(done, ln, ln_new)
        beliefs = jnp.where(done, beliefs, b_new)
        done = jnp.logical_or(done, diff < THRESH)
    return beliefs
